# Initial kernel scaffold; baseline (speedup 1.0000x reference)
import functools

import jax
import jax.numpy as jnp
from jax import lax
from jax.experimental import pallas as pl
from jax.experimental.pallas import tpu as pltpu
from jax.experimental.pallas import tpu_sc as plsc

N_NODES = 10000
D = 128
N_EDGES = 320000

NUM_CORES = 2
NUM_SUBCORES = 16

ROWS_PER_TILE = 632
R = ROWS_PER_TILE * NUM_SUBCORES
TRASH = R - 1

CHUNK = 128
CHUNKS_PER_TILE = 158
EDGES_PER_TILE = CHUNKS_PER_TILE * CHUNK
E_PAD = EDGES_PER_TILE * NUM_SUBCORES

_mesh = plsc.VectorSubcoreMesh(core_axis_name="c", subcore_axis_name="s")


@functools.partial(
    pl.kernel,
    out_type=jax.ShapeDtypeStruct((NUM_CORES, 2, R), jnp.float32),
    mesh=_mesh,
    scratch_types=[
        pltpu.VMEM((CHUNK,), jnp.int32),
        pltpu.VMEM((CHUNK,), jnp.float32),
        pltpu.VMEM((ROWS_PER_TILE,), jnp.float32),
        pltpu.VMEM_SHARED((R,), jnp.float32),
        pltpu.VMEM_SHARED((R,), jnp.float32),
    ],
)
def _degree_kernel(src_hbm, dst_hbm, deg_out, idx_v, ones_v, zero_v, od_s, id_s):
    c = lax.axis_index("c")
    s = lax.axis_index("s")
    base = s * EDGES_PER_TILE

    for k in range(CHUNK // 16):
        ones_v[pl.ds(16 * k, 16)] = jnp.ones((16,), jnp.float32)
    for k in range(ROWS_PER_TILE // 16 + 1):
        off = min(16 * k, ROWS_PER_TILE - 16)
        zero_v[pl.ds(off, 16)] = jnp.zeros((16,), jnp.float32)

    row0 = s * ROWS_PER_TILE
    pltpu.sync_copy(zero_v, od_s.at[pl.ds(row0, ROWS_PER_TILE)])
    pltpu.sync_copy(zero_v, id_s.at[pl.ds(row0, ROWS_PER_TILE)])
    plsc.subcore_barrier()

    def body(i, carry):
        pltpu.sync_copy(src_hbm.at[c, pl.ds(base + CHUNK * i, CHUNK)], idx_v)
        pltpu.sync_copy(ones_v, od_s.at[idx_v], add=True)
        pltpu.sync_copy(dst_hbm.at[c, pl.ds(base + CHUNK * i, CHUNK)], idx_v)
        pltpu.sync_copy(ones_v, id_s.at[idx_v], add=True)
        return carry

    lax.fori_loop(0, CHUNKS_PER_TILE, body, 0)
    plsc.subcore_barrier()

    pltpu.sync_copy(od_s.at[pl.ds(row0, ROWS_PER_TILE)],
                    deg_out.at[c, 0, pl.ds(row0, ROWS_PER_TILE)])
    pltpu.sync_copy(id_s.at[pl.ds(row0, ROWS_PER_TILE)],
                    deg_out.at[c, 1, pl.ds(row0, ROWS_PER_TILE)])


@functools.partial(
    pl.kernel,
    out_type=jax.ShapeDtypeStruct((NUM_CORES, R, D), jnp.float32),
    mesh=_mesh,
    scratch_types=[
        pltpu.VMEM((CHUNK,), jnp.int32),
        pltpu.VMEM((CHUNK,), jnp.int32),
        pltpu.VMEM((CHUNK, D), jnp.float32),
        pltpu.VMEM((CHUNK, D), jnp.float32),
        pltpu.VMEM_SHARED((R, D), jnp.float32),
        pltpu.SemaphoreType.DMA,
    ],
)
def _agg_kernel(featg_hbm, srcg_hbm, dst_hbm, agg_out,
                sidx, didx, rows, zrows, acc, sem):
    c = lax.axis_index("c")
    s = lax.axis_index("s")
    base = s * EDGES_PER_TILE
    row0 = s * ROWS_PER_TILE

    def zbody(k, carry):
        zrows[k // 8, pl.ds(16 * (k % 8), 16)] = jnp.zeros((16,), jnp.float32)
        return carry

    lax.fori_loop(0, CHUNK * (D // 16), zbody, 0)
    for k in range(4):
        pltpu.sync_copy(zrows, acc.at[pl.ds(row0 + CHUNK * k, CHUNK), :])
    pltpu.sync_copy(zrows.at[pl.ds(0, ROWS_PER_TILE - 4 * CHUNK)],
                    acc.at[pl.ds(row0 + 4 * CHUNK, ROWS_PER_TILE - 4 * CHUNK), :])
    plsc.subcore_barrier()

    def body(i, carry):
        pltpu.sync_copy(srcg_hbm.at[c, pl.ds(base + CHUNK * i, CHUNK)], sidx)
        pltpu.sync_copy(dst_hbm.at[c, pl.ds(base + CHUNK * i, CHUNK)], didx)
        pltpu.async_copy(featg_hbm.at[sidx], rows, sem).wait()
        pltpu.sync_copy(rows, acc.at[didx], add=True)
        return carry

    lax.fori_loop(0, CHUNKS_PER_TILE, body, 0)
    plsc.subcore_barrier()

    pltpu.sync_copy(acc.at[pl.ds(row0, ROWS_PER_TILE), :],
                    agg_out.at[c, pl.ds(row0, ROWS_PER_TILE), :])


def _scale_body(x_ref, od_ref, id_ref, feat_ref, sin_ref):
    od = od_ref[0]
    s_out = lax.rsqrt(jnp.maximum(od, 1.0))
    feat_ref[...] = x_ref[...] * s_out
    sin_ref[0] = lax.rsqrt(jnp.maximum(id_ref[0], 1.0))


def _scale_call(x_pad, odg, idg):
    nblk = R // ROWS_PER_TILE
    return pl.pallas_call(
        _scale_body,
        grid=(2, nblk),
        in_specs=[
            pl.BlockSpec((ROWS_PER_TILE, D), lambda h, i: (i, 0)),
            pl.BlockSpec((1, ROWS_PER_TILE, 1), lambda h, i: (h, i, 0)),
            pl.BlockSpec((1, ROWS_PER_TILE, 1), lambda h, i: (h, i, 0)),
        ],
        out_specs=[
            pl.BlockSpec((ROWS_PER_TILE, D), lambda h, i: (h * nblk + i, 0)),
            pl.BlockSpec((1, ROWS_PER_TILE, 1), lambda h, i: (h, i, 0)),
        ],
        out_shape=[
            jax.ShapeDtypeStruct((2 * R, D), jnp.float32),
            jax.ShapeDtypeStruct((2, R, 1), jnp.float32),
        ],
    )(x_pad, odg, idg)


def _out_body(a0_ref, a1_ref, s0_ref, s1_ref, w0_ref, w1_ref, b0_ref, b1_ref,
              y_ref):
    a0 = a0_ref[...] * s0_ref[...]
    a1 = a1_ref[...] * s1_ref[...]
    y = jnp.dot(a0, w0_ref[...], preferred_element_type=jnp.float32)
    y += jnp.dot(a1, w1_ref[...], preferred_element_type=jnp.float32)
    y_ref[...] = y + b0_ref[...] + b1_ref[...]


def _out_call(agg0, agg1, sin0, sin1, W0, W1, b0, b1):
    nblk = R // ROWS_PER_TILE
    return pl.pallas_call(
        _out_body,
        grid=(nblk,),
        in_specs=[
            pl.BlockSpec((ROWS_PER_TILE, D), lambda i: (i, 0)),
            pl.BlockSpec((ROWS_PER_TILE, D), lambda i: (i, 0)),
            pl.BlockSpec((ROWS_PER_TILE, 1), lambda i: (i, 0)),
            pl.BlockSpec((ROWS_PER_TILE, 1), lambda i: (i, 0)),
            pl.BlockSpec((D, D), lambda i: (0, 0)),
            pl.BlockSpec((D, D), lambda i: (0, 0)),
            pl.BlockSpec((1, D), lambda i: (0, 0)),
            pl.BlockSpec((1, D), lambda i: (0, 0)),
        ],
        out_specs=pl.BlockSpec((ROWS_PER_TILE, D), lambda i: (i, 0)),
        out_shape=jax.ShapeDtypeStruct((R, D), jnp.float32),
    )(agg0, agg1, sin0, sin1, W0, W1, b0, b1)


def kernel(x, edge_index_rel0, edge_index_rel1, W0, b0, W1, b1):
    e0 = edge_index_rel0.astype(jnp.int32)
    e1 = edge_index_rel1.astype(jnp.int32)
    pad = ((0, 0), (0, E_PAD - N_EDGES))
    e0 = jnp.pad(e0, pad, constant_values=TRASH)
    e1 = jnp.pad(e1, pad, constant_values=TRASH)
    src_all = jnp.stack([e0[0], e1[0]])
    dst_all = jnp.stack([e0[1], e1[1]])
    srcg = src_all + (jnp.arange(2, dtype=jnp.int32) * R)[:, None]

    x_pad = jnp.pad(x, ((0, R - N_NODES), (0, 0)))

    degs = _degree_kernel(src_all, dst_all)
    odg = degs[:, 0, :].reshape(2, R, 1)
    idg = degs[:, 1, :].reshape(2, R, 1)

    featg, sing = _scale_call(x_pad, odg, idg)

    agg = _agg_kernel(featg, srcg, dst_all)

    y = _out_call(agg[0], agg[1], sing[0], sing[1], W0, W1, b0, b1)
    return y[:N_NODES]


# speedup vs baseline: 4.2873x; 4.2873x over previous
import functools

import jax
import jax.numpy as jnp
from jax import lax
from jax.experimental import pallas as pl
from jax.experimental.pallas import tpu as pltpu
from jax.experimental.pallas import tpu_sc as plsc

N_NODES = 10000
D = 128
N_EDGES = 320000

NUM_CORES = 2
NUM_SUBCORES = 16

ROWS_PER_TILE = 632
R = ROWS_PER_TILE * NUM_SUBCORES
TRASH = R - 1

CHUNK = 128
CHUNKS_PER_TILE = 158
EDGES_PER_TILE = CHUNKS_PER_TILE * CHUNK
E_PAD = EDGES_PER_TILE * NUM_SUBCORES

_mesh = plsc.VectorSubcoreMesh(core_axis_name="c", subcore_axis_name="s")


@functools.partial(
    pl.kernel,
    out_type=jax.ShapeDtypeStruct((NUM_CORES * 2 * R,), jnp.float32),
    mesh=_mesh,
    scratch_types=[
        pltpu.VMEM((CHUNK,), jnp.int32),
        pltpu.VMEM((CHUNK,), jnp.float32),
        pltpu.VMEM((ROWS_PER_TILE,), jnp.float32),
        pltpu.VMEM_SHARED((R,), jnp.float32),
        pltpu.VMEM_SHARED((R,), jnp.float32),
    ],
)
def _degree_kernel(src_hbm, dst_hbm, deg_out, idx_v, ones_v, zero_v, od_s, id_s):
    c = lax.axis_index("c")
    s = lax.axis_index("s")
    base = c * E_PAD + s * EDGES_PER_TILE

    for k in range(CHUNK // 16):
        ones_v[pl.ds(16 * k, 16)] = jnp.ones((16,), jnp.float32)
    for k in range(ROWS_PER_TILE // 16 + 1):
        off = min(16 * k, ROWS_PER_TILE - 16)
        zero_v[pl.ds(off, 16)] = jnp.zeros((16,), jnp.float32)

    row0 = s * ROWS_PER_TILE
    pltpu.sync_copy(zero_v, od_s.at[pl.ds(row0, ROWS_PER_TILE)])
    pltpu.sync_copy(zero_v, id_s.at[pl.ds(row0, ROWS_PER_TILE)])
    plsc.subcore_barrier()

    def body(i, carry):
        pltpu.sync_copy(src_hbm.at[pl.ds(base + CHUNK * i, CHUNK)], idx_v)
        pltpu.sync_copy(ones_v, od_s.at[idx_v], add=True)
        pltpu.sync_copy(dst_hbm.at[pl.ds(base + CHUNK * i, CHUNK)], idx_v)
        pltpu.sync_copy(ones_v, id_s.at[idx_v], add=True)
        return carry

    lax.fori_loop(0, CHUNKS_PER_TILE, body, 0)
    plsc.subcore_barrier()

    pltpu.sync_copy(od_s.at[pl.ds(row0, ROWS_PER_TILE)], zero_v)
    pltpu.sync_copy(zero_v, deg_out.at[pl.ds(2 * c * R + row0, ROWS_PER_TILE)])
    pltpu.sync_copy(id_s.at[pl.ds(row0, ROWS_PER_TILE)], zero_v)
    pltpu.sync_copy(zero_v, deg_out.at[pl.ds((2 * c + 1) * R + row0, ROWS_PER_TILE)])


@functools.partial(
    pl.kernel,
    out_type=jax.ShapeDtypeStruct((NUM_CORES, R, D), jnp.float32),
    mesh=_mesh,
    scratch_types=[
        pltpu.VMEM((CHUNK,), jnp.int32),
        pltpu.VMEM((CHUNK,), jnp.int32),
        pltpu.VMEM((CHUNK, D), jnp.float32),
        pltpu.VMEM((CHUNK, D), jnp.float32),
        pltpu.VMEM_SHARED((R, D), jnp.float32),
        pltpu.SemaphoreType.DMA,
    ],
)
def _agg_kernel(featg_hbm, srcg_hbm, dst_hbm, agg_out,
                sidx, didx, rows, zrows, acc, sem):
    c = lax.axis_index("c")
    s = lax.axis_index("s")
    base = c * E_PAD + s * EDGES_PER_TILE
    row0 = s * ROWS_PER_TILE

    def zbody(k, carry):
        zrows[k // 8, pl.ds(16 * (k % 8), 16)] = jnp.zeros((16,), jnp.float32)
        return carry

    lax.fori_loop(0, CHUNK * (D // 16), zbody, 0)
    for k in range(4):
        pltpu.sync_copy(zrows, acc.at[pl.ds(row0 + CHUNK * k, CHUNK), :])
    pltpu.sync_copy(zrows.at[pl.ds(0, ROWS_PER_TILE - 4 * CHUNK)],
                    acc.at[pl.ds(row0 + 4 * CHUNK, ROWS_PER_TILE - 4 * CHUNK), :])
    plsc.subcore_barrier()

    def body(i, carry):
        pltpu.sync_copy(srcg_hbm.at[pl.ds(base + CHUNK * i, CHUNK)], sidx)
        pltpu.sync_copy(dst_hbm.at[pl.ds(base + CHUNK * i, CHUNK)], didx)
        pltpu.async_copy(featg_hbm.at[sidx], rows, sem).wait()
        pltpu.sync_copy(rows, acc.at[didx], add=True)
        return carry

    lax.fori_loop(0, CHUNKS_PER_TILE, body, 0)
    plsc.subcore_barrier()

    for k in range(4):
        pltpu.sync_copy(acc.at[pl.ds(row0 + CHUNK * k, CHUNK), :], rows)
        pltpu.sync_copy(rows, agg_out.at[c, pl.ds(row0 + CHUNK * k, CHUNK), :])
    tail = ROWS_PER_TILE - 4 * CHUNK
    pltpu.sync_copy(acc.at[pl.ds(row0 + 4 * CHUNK, tail), :],
                    rows.at[pl.ds(0, tail)])
    pltpu.sync_copy(rows.at[pl.ds(0, tail)],
                    agg_out.at[c, pl.ds(row0 + 4 * CHUNK, tail), :])


def _scale_body(x_ref, od_ref, id_ref, feat_ref, sin_ref):
    od = od_ref[0]
    s_out = lax.rsqrt(jnp.maximum(od, 1.0))
    feat_ref[...] = x_ref[...] * s_out
    sin_ref[0] = lax.rsqrt(jnp.maximum(id_ref[0], 1.0))


def _scale_call(x_pad, odg, idg):
    nblk = R // ROWS_PER_TILE
    return pl.pallas_call(
        _scale_body,
        grid=(2, nblk),
        in_specs=[
            pl.BlockSpec((ROWS_PER_TILE, D), lambda h, i: (i, 0)),
            pl.BlockSpec((1, ROWS_PER_TILE, 1), lambda h, i: (h, i, 0)),
            pl.BlockSpec((1, ROWS_PER_TILE, 1), lambda h, i: (h, i, 0)),
        ],
        out_specs=[
            pl.BlockSpec((ROWS_PER_TILE, D), lambda h, i: (h * nblk + i, 0)),
            pl.BlockSpec((1, ROWS_PER_TILE, 1), lambda h, i: (h, i, 0)),
        ],
        out_shape=[
            jax.ShapeDtypeStruct((2 * R, D), jnp.float32),
            jax.ShapeDtypeStruct((2, R, 1), jnp.float32),
        ],
    )(x_pad, odg, idg)


def _out_body(a0_ref, a1_ref, s0_ref, s1_ref, w0_ref, w1_ref, b0_ref, b1_ref,
              y_ref):
    a0 = a0_ref[...] * s0_ref[...]
    a1 = a1_ref[...] * s1_ref[...]
    y = jnp.dot(a0, w0_ref[...], preferred_element_type=jnp.float32)
    y += jnp.dot(a1, w1_ref[...], preferred_element_type=jnp.float32)
    y_ref[...] = y + b0_ref[...] + b1_ref[...]


def _out_call(agg0, agg1, sin0, sin1, W0, W1, b0, b1):
    nblk = R // ROWS_PER_TILE
    return pl.pallas_call(
        _out_body,
        grid=(nblk,),
        in_specs=[
            pl.BlockSpec((ROWS_PER_TILE, D), lambda i: (i, 0)),
            pl.BlockSpec((ROWS_PER_TILE, D), lambda i: (i, 0)),
            pl.BlockSpec((ROWS_PER_TILE, 1), lambda i: (i, 0)),
            pl.BlockSpec((ROWS_PER_TILE, 1), lambda i: (i, 0)),
            pl.BlockSpec((D, D), lambda i: (0, 0)),
            pl.BlockSpec((D, D), lambda i: (0, 0)),
            pl.BlockSpec((1, D), lambda i: (0, 0)),
            pl.BlockSpec((1, D), lambda i: (0, 0)),
        ],
        out_specs=pl.BlockSpec((ROWS_PER_TILE, D), lambda i: (i, 0)),
        out_shape=jax.ShapeDtypeStruct((R, D), jnp.float32),
    )(agg0, agg1, sin0, sin1, W0, W1, b0, b1)


def kernel(x, edge_index_rel0, edge_index_rel1, W0, b0, W1, b1):
    e0 = edge_index_rel0.astype(jnp.int32)
    e1 = edge_index_rel1.astype(jnp.int32)
    pad = ((0, 0), (0, E_PAD - N_EDGES))
    e0 = jnp.pad(e0, pad, constant_values=TRASH)
    e1 = jnp.pad(e1, pad, constant_values=TRASH)
    src_all = jnp.concatenate([e0[0], e1[0]])
    dst_all = jnp.concatenate([e0[1], e1[1]])
    srcg = src_all + jnp.repeat(
        jnp.arange(2, dtype=jnp.int32) * R, E_PAD, total_repeat_length=2 * E_PAD)

    x_pad = jnp.pad(x, ((0, R - N_NODES), (0, 0)))

    degs = _degree_kernel(src_all, dst_all).reshape(2, 2, R)
    odg = degs[:, 0, :].reshape(2, R, 1)
    idg = degs[:, 1, :].reshape(2, R, 1)

    featg, sing = _scale_call(x_pad, odg, idg)

    agg = _agg_kernel(featg, srcg, dst_all)

    y = _out_call(agg[0], agg[1], sing[0], sing[1], W0, W1,
                  b0.reshape(1, D), b1.reshape(1, D))
    return y[:N_NODES]


# speedup vs baseline: 5.3388x; 1.2453x over previous
import functools

import jax
import jax.numpy as jnp
from jax import lax
from jax.experimental import pallas as pl
from jax.experimental.pallas import tpu as pltpu
from jax.experimental.pallas import tpu_sc as plsc

N_NODES = 10000
D = 128
N_EDGES = 320000

NUM_CORES = 2
NUM_SUBCORES = 16

ROWS_PER_TILE = 632
R = ROWS_PER_TILE * NUM_SUBCORES
TRASH = R - 1

CHUNK = 128
CHUNKS_PER_TILE = 160
EDGES_PER_TILE = CHUNKS_PER_TILE * CHUNK
E_PAD = EDGES_PER_TILE * NUM_SUBCORES
SLAB_ROWS = NUM_CORES * NUM_SUBCORES * CHUNKS_PER_TILE

FIRE = 16

_mesh = plsc.VectorSubcoreMesh(core_axis_name="c", subcore_axis_name="s")


def _fill(ref, n, value):
    def body(k, carry):
        ref[pl.ds(16 * k, 16)] = jnp.full((16,), value, jnp.float32)
        return carry
    lax.fori_loop(0, n // 16, body, 0)


@functools.partial(
    pl.kernel,
    out_type=jax.ShapeDtypeStruct((NUM_CORES * R,), jnp.float32),
    mesh=_mesh,
    scratch_types=[
        pltpu.VMEM((CHUNKS_PER_TILE, CHUNK), jnp.int32),
        pltpu.VMEM((CHUNK,), jnp.float32),
        pltpu.VMEM((ROWS_PER_TILE,), jnp.float32),
        pltpu.VMEM_SHARED((R,), jnp.float32),
        pltpu.SemaphoreType.DMA,
    ],
)
def _degree_kernel(src_hbm, deg_out, idx_all, ones_v, stage_v, od_s, sem):
    c = lax.axis_index("c")
    s = lax.axis_index("s")
    slab0 = (c * NUM_SUBCORES + s) * CHUNKS_PER_TILE
    row0 = s * ROWS_PER_TILE

    pltpu.sync_copy(src_hbm.at[pl.ds(slab0, CHUNKS_PER_TILE), :], idx_all)
    _fill(ones_v, CHUNK, 1.0)
    _fill(stage_v, ROWS_PER_TILE, 0.0)
    pltpu.sync_copy(stage_v, od_s.at[pl.ds(row0, ROWS_PER_TILE)])
    plsc.subcore_barrier()

    def body(j, carry):
        hs = [
            pltpu.async_copy(ones_v, od_s.at[idx_all.at[FIRE * j + f]], sem,
                             add=True)
            for f in range(FIRE)
        ]
        for h in hs:
            h.wait()
        return carry

    lax.fori_loop(0, CHUNKS_PER_TILE // FIRE, body, 0)
    plsc.subcore_barrier()

    pltpu.sync_copy(od_s.at[pl.ds(row0, ROWS_PER_TILE)], stage_v)
    pltpu.sync_copy(stage_v, deg_out.at[pl.ds(c * R + row0, ROWS_PER_TILE)])


N_QUARTERS = 4
Q_CHUNKS = CHUNKS_PER_TILE // N_QUARTERS


@functools.partial(
    pl.kernel,
    out_type=[
        jax.ShapeDtypeStruct((NUM_CORES, R, D), jnp.float32),
        jax.ShapeDtypeStruct((NUM_CORES * R,), jnp.float32),
    ],
    mesh=_mesh,
    scratch_types=[
        pltpu.VMEM((2 * Q_CHUNKS, CHUNK), jnp.int32),
        pltpu.VMEM((CHUNK, D), jnp.float32),
        pltpu.VMEM((CHUNK, D), jnp.float32),
        pltpu.VMEM((CHUNK,), jnp.float32),
        pltpu.VMEM((ROWS_PER_TILE,), jnp.float32),
        pltpu.VMEM_SHARED((R, D), jnp.float32),
        pltpu.VMEM_SHARED((R,), jnp.float32),
        pltpu.SemaphoreType.DMA,
        pltpu.SemaphoreType.DMA,
        pltpu.SemaphoreType.DMA,
    ],
)
def _agg_kernel(featg_hbm, comb_hbm, agg_out, indeg_out,
                comb_q, rows0, rows1, ones_v, stage_v,
                acc, id_s, sem0, sem1, sem_h):
    c = lax.axis_index("c")
    s = lax.axis_index("s")
    slab0 = (c * NUM_SUBCORES + s) * CHUNKS_PER_TILE * 2
    row0 = s * ROWS_PER_TILE
    rows = (rows0, rows1)
    sems = (sem0, sem1)
    tail = ROWS_PER_TILE - 4 * CHUNK

    def zbody(k, carry):
        rows0[k // 8, pl.ds(16 * (k % 8), 16)] = jnp.zeros((16,), jnp.float32)
        return carry

    lax.fori_loop(0, CHUNK * (D // 16), zbody, 0)
    _fill(ones_v, CHUNK, 1.0)
    _fill(stage_v, ROWS_PER_TILE, 0.0)

    for k in range(4):
        pltpu.sync_copy(rows0, acc.at[pl.ds(row0 + CHUNK * k, CHUNK), :])
    pltpu.sync_copy(rows0.at[pl.ds(0, tail)],
                    acc.at[pl.ds(row0 + 4 * CHUNK, tail), :])
    pltpu.sync_copy(stage_v, id_s.at[pl.ds(row0, ROWS_PER_TILE)])
    plsc.subcore_barrier()

    for q in range(N_QUARTERS):
        pltpu.sync_copy(
            comb_hbm.at[pl.ds(slab0 + 2 * Q_CHUNKS * q, 2 * Q_CHUNKS), :],
            comb_q)

        pltpu.async_copy(featg_hbm.at[comb_q.at[0]], rows0, sem0)
        pltpu.async_copy(featg_hbm.at[comb_q.at[2]], rows1, sem1)

        def body(g, carry):
            for b in range(2):
                k = 2 * g + b
                pltpu.make_async_copy(
                    featg_hbm.at[pl.ds(0, CHUNK), :], rows[b], sems[b]).wait()
                pltpu.sync_copy(rows[b], acc.at[comb_q.at[2 * k + 1]],
                                add=True)

                @pl.when(k + 2 < Q_CHUNKS)
                def _():
                    pltpu.async_copy(
                        featg_hbm.at[comb_q.at[2 * k + 4]], rows[b], sems[b])
            return carry

        lax.fori_loop(0, Q_CHUNKS // 2, body, 0)

        def hbody(j, carry):
            hs = [
                pltpu.async_copy(
                    ones_v, id_s.at[comb_q.at[2 * (8 * j + f) + 1]],
                    sem_h, add=True)
                for f in range(8)
            ]
            for h in hs:
                h.wait()
            return carry

        lax.fori_loop(0, Q_CHUNKS // 8, hbody, 0)
    plsc.subcore_barrier()

    for k in range(4):
        pltpu.sync_copy(acc.at[pl.ds(row0 + CHUNK * k, CHUNK), :], rows0)
        pltpu.sync_copy(rows0, agg_out.at[c, pl.ds(row0 + CHUNK * k, CHUNK), :])
    pltpu.sync_copy(acc.at[pl.ds(row0 + 4 * CHUNK, tail), :],
                    rows0.at[pl.ds(0, tail)])
    pltpu.sync_copy(rows0.at[pl.ds(0, tail)],
                    agg_out.at[c, pl.ds(row0 + 4 * CHUNK, tail), :])
    pltpu.sync_copy(id_s.at[pl.ds(row0, ROWS_PER_TILE)], stage_v)
    pltpu.sync_copy(stage_v, indeg_out.at[pl.ds(c * R + row0, ROWS_PER_TILE)])


def _pack_edges(e0, e1):
    src2d = jnp.stack([e0[0], e1[0] + R]).reshape(SLAB_ROWS, CHUNK)
    dst2d = jnp.stack([e0[1], e1[1]]).reshape(SLAB_ROWS, CHUNK)
    return jnp.stack([src2d, dst2d], axis=1).reshape(2 * SLAB_ROWS, CHUNK)


def _scale_body(x_ref, od_ref, feat_ref):
    s_out = lax.rsqrt(jnp.maximum(od_ref[0], 1.0))
    feat_ref[...] = x_ref[...] * s_out


def _scale_call(x_pad, odg):
    nblk = R // ROWS_PER_TILE
    return pl.pallas_call(
        _scale_body,
        grid=(2, nblk),
        in_specs=[
            pl.BlockSpec((ROWS_PER_TILE, D), lambda h, i: (i, 0)),
            pl.BlockSpec((1, ROWS_PER_TILE, 1), lambda h, i: (h, i, 0)),
        ],
        out_specs=pl.BlockSpec((ROWS_PER_TILE, D), lambda h, i: (h * nblk + i, 0)),
        out_shape=jax.ShapeDtypeStruct((2 * R, D), jnp.float32),
    )(x_pad, odg)


def _out_body(a0_ref, a1_ref, i0_ref, i1_ref, w0_ref, w1_ref, b0_ref, b1_ref,
              y_ref):
    s0 = lax.rsqrt(jnp.maximum(i0_ref[...], 1.0))
    s1 = lax.rsqrt(jnp.maximum(i1_ref[...], 1.0))
    a0 = a0_ref[...] * s0
    a1 = a1_ref[...] * s1
    y = jnp.dot(a0, w0_ref[...], preferred_element_type=jnp.float32)
    y += jnp.dot(a1, w1_ref[...], preferred_element_type=jnp.float32)
    y_ref[...] = y + b0_ref[...] + b1_ref[...]


def _out_call(agg0, agg1, ind0, ind1, W0, W1, b0, b1):
    nblk = R // ROWS_PER_TILE
    return pl.pallas_call(
        _out_body,
        grid=(nblk,),
        in_specs=[
            pl.BlockSpec((ROWS_PER_TILE, D), lambda i: (i, 0)),
            pl.BlockSpec((ROWS_PER_TILE, D), lambda i: (i, 0)),
            pl.BlockSpec((ROWS_PER_TILE, 1), lambda i: (i, 0)),
            pl.BlockSpec((ROWS_PER_TILE, 1), lambda i: (i, 0)),
            pl.BlockSpec((D, D), lambda i: (0, 0)),
            pl.BlockSpec((D, D), lambda i: (0, 0)),
            pl.BlockSpec((1, D), lambda i: (0, 0)),
            pl.BlockSpec((1, D), lambda i: (0, 0)),
        ],
        out_specs=pl.BlockSpec((ROWS_PER_TILE, D), lambda i: (i, 0)),
        out_shape=jax.ShapeDtypeStruct((R, D), jnp.float32),
    )(agg0, agg1, ind0, ind1, W0, W1, b0, b1)


def kernel(x, edge_index_rel0, edge_index_rel1, W0, b0, W1, b1):
    e0 = edge_index_rel0.astype(jnp.int32)
    e1 = edge_index_rel1.astype(jnp.int32)
    pad = ((0, 0), (0, E_PAD - N_EDGES))
    e0 = jnp.pad(e0, pad, constant_values=TRASH)
    e1 = jnp.pad(e1, pad, constant_values=TRASH)
    src2d = jnp.stack([e0[0], e1[0]]).reshape(SLAB_ROWS, CHUNK)
    comb = _pack_edges(e0, e1)

    x_pad = jnp.pad(x, ((0, R - N_NODES), (0, 0)))

    odeg = _degree_kernel(src2d)
    odg = odeg.reshape(2, R, 1)

    featg = _scale_call(x_pad, odg)

    agg, indeg = _agg_kernel(featg, comb)
    ind = indeg.reshape(2, R, 1)

    y = _out_call(agg[0], agg[1], ind[0], ind[1], W0, W1,
                  b0.reshape(1, D), b1.reshape(1, D))
    return y[:N_NODES]
